# Initial kernel scaffold; baseline (speedup 1.0000x reference)
import jax
import jax.numpy as jnp
from jax.experimental import pallas as pl


def kernel(x, edge_index, W1l, W1r, b1, W2l, W2r, b2, Wc, bc):
    raise NotImplementedError("write your pallas kernel here")


# speedup vs baseline: 3.4464x; 3.4464x over previous
import functools

import jax
import jax.numpy as jnp
from jax import lax
from jax.experimental import pallas as pl
from jax.experimental.pallas import tpu as pltpu
from jax.experimental.pallas import tpu_sc as plsc

N = 10000
D_IN = 128
D_H1 = 128
D_H2 = 64
E = 320000

NC = 2
NS = 16
CW = 128
GB = 16
NGROUP = 5
NCHUNK = GB * NGROUP
EPAD = NC * NS * NCHUNK * CW
SINK = N
NPAD = 10240
DEGW = 16
ROWS_PER_TILE = NPAD // NS

BLK = 1000


def _fill(ref, rows, cols, value):
    vec = jnp.full((16,), value, dtype=jnp.float32)

    def body(r, _):
        for k in range(cols // 16):
            ref[r, pl.ds(k * 16, 16)] = vec
        return 0

    lax.fori_loop(0, rows, body, 0, unroll=False)


def _make_spmm(D, with_deg):
    mesh = plsc.VectorSubcoreMesh(core_axis_name="c", subcore_axis_name="s")

    out_type = [jax.ShapeDtypeStruct((NC * NPAD, D), jnp.float32)]
    scratch = [
        pltpu.VMEM((GB, CW), jnp.int32),
        pltpu.VMEM((GB, CW), jnp.int32),
        pltpu.VMEM((CW, D), jnp.float32),
        pltpu.VMEM_SHARED((NPAD, D), jnp.float32),
        pltpu.SemaphoreType.DMA,
    ]
    if with_deg:
        out_type.append(jax.ShapeDtypeStruct((NC * NPAD, DEGW), jnp.float32))
        scratch += [
            pltpu.VMEM((CW, DEGW), jnp.float32),
            pltpu.VMEM_SHARED((NPAD, DEGW), jnp.float32),
        ]

    def body(y_hbm, srcs_hbm, dsts_hbm, *rest):
        if with_deg:
            (out_hbm, deg_hbm, src_v, dst_v, rows_v, acc, gsem,
             ones_v, dacc) = rest
        else:
            out_hbm, src_v, dst_v, rows_v, acc, gsem = rest
        cid = lax.axis_index("c")
        sid = lax.axis_index("s")
        wid = cid * NS + sid

        _fill(rows_v, CW, D, 0.0)
        z0 = sid * (NPAD // NS)
        for q in range((NPAD // NS) // CW):
            pltpu.sync_copy(rows_v, acc.at[pl.ds(z0 + q * CW, CW)])
        if with_deg:
            _fill(ones_v, CW, DEGW, 0.0)
            for q in range((NPAD // NS) // CW):
                pltpu.sync_copy(ones_v, dacc.at[pl.ds(z0 + q * CW, CW)])
            _fill(ones_v, CW, DEGW, 1.0)

        plsc.subcore_barrier()

        def group(g, _):
            i0 = wid * NCHUNK + g * GB
            pltpu.sync_copy(srcs_hbm.at[pl.ds(i0, GB)], src_v)
            pltpu.sync_copy(dsts_hbm.at[pl.ds(i0, GB)], dst_v)

            def chunk(b, _):
                pltpu.async_copy(y_hbm.at[src_v.at[b]], rows_v, gsem).wait()
                pltpu.sync_copy(rows_v, acc.at[dst_v.at[b]], add=True)
                if with_deg:
                    pltpu.sync_copy(ones_v, dacc.at[dst_v.at[b]], add=True)
                return 0

            lax.fori_loop(0, GB, chunk, 0, unroll=False)
            return 0

        lax.fori_loop(0, NGROUP, group, 0, unroll=False)
        plsc.subcore_barrier()

        t0 = sid * ROWS_PER_TILE
        for q in range(ROWS_PER_TILE // CW):
            pltpu.sync_copy(acc.at[pl.ds(t0 + q * CW, CW)], rows_v)
            pltpu.sync_copy(rows_v,
                            out_hbm.at[pl.ds(cid * NPAD + t0 + q * CW, CW)])
        if with_deg:
            for q in range(ROWS_PER_TILE // CW):
                pltpu.sync_copy(dacc.at[pl.ds(t0 + q * CW, CW)], ones_v)
                pltpu.sync_copy(
                    ones_v, deg_hbm.at[pl.ds(cid * NPAD + t0 + q * CW, CW)])

    return pl.kernel(
        body, out_type=out_type, mesh=mesh, scratch_types=scratch,
        compiler_params=pltpu.CompilerParams(use_tc_tiling_on_sc=False))


_spmm_deg = _make_spmm(D_H1, True)
_spmm = _make_spmm(D_H1, False)


def _pre_body(x_ref, wl_ref, wr_ref, b1_ref, y1_ref, z1_ref):
    x = x_ref[...]
    y1_ref[...] = jnp.dot(x, wl_ref[...], preferred_element_type=jnp.float32)
    z1_ref[...] = (jnp.dot(x, wr_ref[...], preferred_element_type=jnp.float32)
                   + b1_ref[...])


def _pre(x, W1l, W1r, b1):
    return pl.pallas_call(
        _pre_body,
        grid=(N // BLK,),
        in_specs=[
            pl.BlockSpec((BLK, D_IN), lambda i: (i, 0)),
            pl.BlockSpec((D_IN, D_H1), lambda i: (0, 0)),
            pl.BlockSpec((D_IN, D_H1), lambda i: (0, 0)),
            pl.BlockSpec((1, D_H1), lambda i: (0, 0)),
        ],
        out_specs=[
            pl.BlockSpec((BLK, D_H1), lambda i: (i, 0)),
            pl.BlockSpec((BLK, D_H1), lambda i: (i, 0)),
        ],
        out_shape=[
            jax.ShapeDtypeStruct((N, D_H1), jnp.float32),
            jax.ShapeDtypeStruct((N, D_H1), jnp.float32),
        ],
    )(x, W1l, W1r, b1)


def _mid_body(s1a_ref, s1b_ref, dega_ref, degb_ref, z1_ref, w2r_ref,
              b2_ref, h1_ref, z2_ref):
    s1 = s1a_ref[...] + s1b_ref[...]
    deg = dega_ref[:, 0:1] + degb_ref[:, 0:1]
    rdeg = 1.0 / jnp.maximum(deg, 1.0)
    h1 = jnp.maximum(s1 * rdeg + z1_ref[...], 0.0)
    h1_ref[...] = h1
    z2_ref[...] = (jnp.dot(h1, w2r_ref[...], preferred_element_type=jnp.float32)
                   + b2_ref[...])


def _mid(s1a, s1b, dega, degb, z1, W2r, b2):
    return pl.pallas_call(
        _mid_body,
        grid=(N // BLK,),
        in_specs=[
            pl.BlockSpec((BLK, D_H1), lambda i: (i, 0)),
            pl.BlockSpec((BLK, D_H1), lambda i: (i, 0)),
            pl.BlockSpec((BLK, DEGW), lambda i: (i, 0)),
            pl.BlockSpec((BLK, DEGW), lambda i: (i, 0)),
            pl.BlockSpec((BLK, D_H1), lambda i: (i, 0)),
            pl.BlockSpec((D_H1, D_H2), lambda i: (0, 0)),
            pl.BlockSpec((1, D_H2), lambda i: (0, 0)),
        ],
        out_specs=[
            pl.BlockSpec((BLK, D_H1), lambda i: (i, 0)),
            pl.BlockSpec((BLK, D_H2), lambda i: (i, 0)),
        ],
        out_shape=[
            jax.ShapeDtypeStruct((N, D_H1), jnp.float32),
            jax.ShapeDtypeStruct((N, D_H2), jnp.float32),
        ],
    )(s1a, s1b, dega, degb, z1, W2r, b2)


def _post_body(s2a_ref, s2b_ref, dega_ref, degb_ref, z2_ref, w2l_ref, wc_ref,
               bc_ref, out_ref):
    s2 = s2a_ref[...] + s2b_ref[...]
    deg = dega_ref[:, 0:1] + degb_ref[:, 0:1]
    rdeg = 1.0 / jnp.maximum(deg, 1.0)
    mean2 = s2 * rdeg
    h2 = jnp.maximum(
        jnp.dot(mean2, w2l_ref[...], preferred_element_type=jnp.float32)
        + z2_ref[...], 0.0)
    logits = (jnp.dot(h2, wc_ref[...], preferred_element_type=jnp.float32)
              + bc_ref[...])
    l0 = logits[:, 0:1]
    l1 = logits[:, 1:2]
    m = jnp.maximum(l0, l1)
    lse = m + jnp.log(jnp.exp(l0 - m) + jnp.exp(l1 - m))
    out_ref[...] = jnp.concatenate([l0 - lse, l1 - lse], axis=1)


def _post(s2a, s2b, dega, degb, z2, W2l, Wc, bc):
    return pl.pallas_call(
        _post_body,
        grid=(N // BLK,),
        in_specs=[
            pl.BlockSpec((BLK, D_H1), lambda i: (i, 0)),
            pl.BlockSpec((BLK, D_H1), lambda i: (i, 0)),
            pl.BlockSpec((BLK, DEGW), lambda i: (i, 0)),
            pl.BlockSpec((BLK, DEGW), lambda i: (i, 0)),
            pl.BlockSpec((BLK, D_H2), lambda i: (i, 0)),
            pl.BlockSpec((D_H1, D_H2), lambda i: (0, 0)),
            pl.BlockSpec((D_H2, 128), lambda i: (0, 0)),
            pl.BlockSpec((1, 128), lambda i: (0, 0)),
        ],
        out_specs=pl.BlockSpec((BLK, 2), lambda i: (i, 0)),
        out_shape=jax.ShapeDtypeStruct((N, 2), jnp.float32),
    )(s2a, s2b, dega, degb, z2, W2l, Wc, bc)


def kernel(x, edge_index, W1l, W1r, b1, W2l, W2r, b2, Wc, bc):
    src = edge_index[0].astype(jnp.int32)
    dst = edge_index[1].astype(jnp.int32)
    pad = EPAD - E
    srcs = jnp.concatenate(
        [src, jnp.zeros((pad,), jnp.int32)]).reshape(NC * NS * NCHUNK, CW)
    dsts = jnp.concatenate(
        [dst, jnp.full((pad,), SINK, jnp.int32)]).reshape(NC * NS * NCHUNK, CW)

    wc_pad = jnp.zeros((D_H2, 128), jnp.float32).at[:, :2].set(Wc)
    bc_pad = jnp.zeros((1, 128), jnp.float32).at[0, :2].set(bc)

    y1, z1 = _pre(x, W1l, W1r, b1.reshape(1, -1))
    s1p, degp = _spmm_deg(y1, srcs, dsts)
    s1a, s1b = s1p[:N], s1p[NPAD:NPAD + N]
    dega, degb = degp[:N], degp[NPAD:NPAD + N]
    h1, z2 = _mid(s1a, s1b, dega, degb, z1, W2r, b2.reshape(1, -1))
    (s2p,) = _spmm(h1, srcs, dsts)
    return _post(s2p[:N], s2p[NPAD:NPAD + N], dega, degb, z2, W2l,
                 wc_pad, bc_pad)


# speedup vs baseline: 4.3075x; 1.2499x over previous
import jax
import jax.numpy as jnp
from jax import lax
from jax.experimental import pallas as pl
from jax.experimental.pallas import tpu as pltpu
from jax.experimental.pallas import tpu_sc as plsc

N = 10000
D_IN = 128
D_H1 = 128
D_H2 = 64
E = 320000

NC = 2
NS = 16
NW = NC * NS
CW = 128
GB = 8
NGROUP = 10
NCHUNK = GB * NGROUP
EPAD = NW * NCHUNK * CW
SINK = N
NPAD = 10016
ZPT = NPAD // NS
DRT = N // NS

BLK = 1000


def _fill_zero(ref, rows, cols):
    vec = jnp.zeros((16,), dtype=jnp.float32)

    def body(r, _):
        for k in range(cols // 16):
            ref[r, pl.ds(k * 16, 16)] = vec
        return 0

    lax.fori_loop(0, rows, body, 0, unroll=False)


def _make_spmm(D, with_deg):
    mesh = plsc.VectorSubcoreMesh(core_axis_name="c", subcore_axis_name="s")

    out_type = [jax.ShapeDtypeStruct((NC * N, D), jnp.float32)]
    scratch = [
        pltpu.VMEM((GB, CW), jnp.int32),
        pltpu.VMEM((GB, CW), jnp.int32),
        pltpu.VMEM((CW, D), jnp.float32),
        pltpu.VMEM((CW, D), jnp.float32),
        pltpu.VMEM_SHARED((NPAD, D), jnp.float32),
        pltpu.SemaphoreType.DMA,
        pltpu.SemaphoreType.DMA,
    ]
    if with_deg:
        out_type.append(jax.ShapeDtypeStruct((NW * NPAD,), jnp.float32))
        scratch.append(pltpu.VMEM((NPAD,), jnp.float32))

    def body(y_hbm, srcs_hbm, dsts_hbm, *rest):
        if with_deg:
            (out_hbm, deg_hbm, src_v, dst_v, rows_a, rows_b, acc,
             sem_a, sem_b, hist) = rest
        else:
            (out_hbm, src_v, dst_v, rows_a, rows_b, acc,
             sem_a, sem_b) = rest
        cid = lax.axis_index("c")
        sid = lax.axis_index("s")
        wid = cid * NS + sid
        bufs = (rows_a, rows_b)
        sems = (sem_a, sem_b)
        ones16 = jnp.full((16,), 1.0, dtype=jnp.float32)

        _fill_zero(rows_a, CW, D)
        z0 = sid * ZPT
        zoff = 0
        while zoff < ZPT:
            zn = min(CW, ZPT - zoff)
            pltpu.sync_copy(rows_a.at[pl.ds(0, zn)],
                            acc.at[pl.ds(z0 + zoff, zn)])
            zoff += zn
        if with_deg:
            zv = jnp.zeros((16,), dtype=jnp.float32)

            def zrow(r, _):
                hist[pl.ds(r * 16, 16)] = zv
                return 0

            lax.fori_loop(0, NPAD // 16, zrow, 0, unroll=False)
        plsc.subcore_barrier()

        def group(g, _):
            i0 = wid * NCHUNK + g * GB
            pltpu.sync_copy(srcs_hbm.at[pl.ds(i0, GB)], src_v)
            pltpu.sync_copy(dsts_hbm.at[pl.ds(i0, GB)], dst_v)
            cp = pltpu.async_copy(y_hbm.at[src_v.at[0]], rows_a, sem_a)
            for b in range(GB):
                buf = bufs[b % 2]
                cp.wait()
                if b + 1 < GB:
                    cp = pltpu.async_copy(y_hbm.at[src_v.at[b + 1]],
                                          bufs[(b + 1) % 2],
                                          sems[(b + 1) % 2])
                if with_deg:
                    for l in range(CW // 16):
                        idx = dst_v[b, pl.ds(l * 16, 16)]
                        plsc.addupdate_scatter(hist, [idx], ones16)
                pltpu.sync_copy(buf, acc.at[dst_v.at[b]], add=True)
            return 0

        lax.fori_loop(0, NGROUP, group, 0, unroll=False)
        plsc.subcore_barrier()

        t0 = sid * DRT
        for q in range(5):
            pltpu.sync_copy(acc.at[pl.ds(t0 + q * 125, 125)],
                            rows_a.at[pl.ds(0, 125)])
            pltpu.sync_copy(rows_a.at[pl.ds(0, 125)],
                            out_hbm.at[pl.ds(cid * N + t0 + q * 125, 125)])
        if with_deg:
            pltpu.sync_copy(hist, deg_hbm.at[pl.ds(wid * NPAD, NPAD)])

    return pl.kernel(
        body, out_type=out_type, mesh=mesh, scratch_types=scratch,
        compiler_params=pltpu.CompilerParams(use_tc_tiling_on_sc=False,
                                             needs_layout_passes=False))


_spmm_deg = _make_spmm(D_H1, True)
_spmm = _make_spmm(D_H2, False)


def _pre_body(x_ref, wl_ref, wr_ref, b1_ref, y1_ref, z1_ref):
    x = x_ref[...]
    y1_ref[...] = jnp.dot(x, wl_ref[...], preferred_element_type=jnp.float32)
    z1_ref[...] = (jnp.dot(x, wr_ref[...], preferred_element_type=jnp.float32)
                   + b1_ref[...])


def _pre(x, W1l, W1r, b1):
    return pl.pallas_call(
        _pre_body,
        grid=(N // BLK,),
        in_specs=[
            pl.BlockSpec((BLK, D_IN), lambda i: (i, 0)),
            pl.BlockSpec((D_IN, D_H1), lambda i: (0, 0)),
            pl.BlockSpec((D_IN, D_H1), lambda i: (0, 0)),
            pl.BlockSpec((1, D_H1), lambda i: (0, 0)),
        ],
        out_specs=[
            pl.BlockSpec((BLK, D_H1), lambda i: (i, 0)),
            pl.BlockSpec((BLK, D_H1), lambda i: (i, 0)),
        ],
        out_shape=[
            jax.ShapeDtypeStruct((N, D_H1), jnp.float32),
            jax.ShapeDtypeStruct((N, D_H1), jnp.float32),
        ],
    )(x, W1l, W1r, b1)


def _rdeg(degt_ref):
    deg = jnp.sum(degt_ref[...], axis=1, keepdims=True)
    return 1.0 / jnp.maximum(deg, 1.0)


def _mid_body(s1a_ref, s1b_ref, degt_ref, z1_ref, w2l_ref, w2r_ref,
              b2_ref, y2_ref, z2_ref):
    s1 = s1a_ref[...] + s1b_ref[...]
    h1 = jnp.maximum(s1 * _rdeg(degt_ref) + z1_ref[...], 0.0)
    y2_ref[...] = jnp.dot(h1, w2l_ref[...], preferred_element_type=jnp.float32)
    z2_ref[...] = (jnp.dot(h1, w2r_ref[...], preferred_element_type=jnp.float32)
                   + b2_ref[...])


def _mid(s1p, degt, z1, W2l, W2r, b2):
    return pl.pallas_call(
        _mid_body,
        grid=(N // BLK,),
        in_specs=[
            pl.BlockSpec((BLK, D_H1), lambda i: (i, 0)),
            pl.BlockSpec((BLK, D_H1), lambda i: (i + N // BLK, 0)),
            pl.BlockSpec((BLK, NW), lambda i: (i, 0)),
            pl.BlockSpec((BLK, D_H1), lambda i: (i, 0)),
            pl.BlockSpec((D_H1, D_H2), lambda i: (0, 0)),
            pl.BlockSpec((D_H1, D_H2), lambda i: (0, 0)),
            pl.BlockSpec((1, D_H2), lambda i: (0, 0)),
        ],
        out_specs=[
            pl.BlockSpec((BLK, D_H2), lambda i: (i, 0)),
            pl.BlockSpec((BLK, D_H2), lambda i: (i, 0)),
        ],
        out_shape=[
            jax.ShapeDtypeStruct((N, D_H2), jnp.float32),
            jax.ShapeDtypeStruct((N, D_H2), jnp.float32),
        ],
    )(s1p, s1p, degt, z1, W2l, W2r, b2)


def _post_body(s2a_ref, s2b_ref, degt_ref, z2_ref, wc_ref, bc_ref, out_ref):
    mean2 = (s2a_ref[...] + s2b_ref[...]) * _rdeg(degt_ref)
    h2 = jnp.maximum(mean2 + z2_ref[...], 0.0)
    logits = (jnp.dot(h2, wc_ref[...], preferred_element_type=jnp.float32)
              + bc_ref[...])
    l0 = logits[:, 0:1]
    l1 = logits[:, 1:2]
    m = jnp.maximum(l0, l1)
    lse = m + jnp.log(jnp.exp(l0 - m) + jnp.exp(l1 - m))
    out_ref[...] = jnp.concatenate([l0 - lse, l1 - lse], axis=1)


def _post(s2p, degt, z2, Wc, bc):
    return pl.pallas_call(
        _post_body,
        grid=(N // BLK,),
        in_specs=[
            pl.BlockSpec((BLK, D_H2), lambda i: (i, 0)),
            pl.BlockSpec((BLK, D_H2), lambda i: (i + N // BLK, 0)),
            pl.BlockSpec((BLK, NW), lambda i: (i, 0)),
            pl.BlockSpec((BLK, D_H2), lambda i: (i, 0)),
            pl.BlockSpec((D_H2, 128), lambda i: (0, 0)),
            pl.BlockSpec((1, 128), lambda i: (0, 0)),
        ],
        out_specs=pl.BlockSpec((BLK, 2), lambda i: (i, 0)),
        out_shape=jax.ShapeDtypeStruct((N, 2), jnp.float32),
    )(s2p, s2p, degt, z2, Wc, bc)


def kernel(x, edge_index, W1l, W1r, b1, W2l, W2r, b2, Wc, bc):
    src = edge_index[0].astype(jnp.int32)
    dst = edge_index[1].astype(jnp.int32)
    pad = EPAD - E
    srcs = jnp.concatenate(
        [src, jnp.zeros((pad,), jnp.int32)]).reshape(NW * NCHUNK, CW)
    dsts = jnp.concatenate(
        [dst, jnp.full((pad,), SINK, jnp.int32)]).reshape(NW * NCHUNK, CW)

    wc_pad = jnp.zeros((D_H2, 128), jnp.float32).at[:, :2].set(Wc)
    bc_pad = jnp.zeros((1, 128), jnp.float32).at[0, :2].set(bc)

    y1, z1 = _pre(x, W1l, W1r, b1.reshape(1, -1))
    s1p, deg_flat = _spmm_deg(y1, srcs, dsts)
    degt = deg_flat.reshape(NW, NPAD).T
    y2, z2 = _mid(s1p, degt, z1, W2l, W2r, b2.reshape(1, -1))
    (s2p,) = _spmm(y2, srcs, dsts)
    return _post(s2p, degt, z2, wc_pad, bc_pad)


# speedup vs baseline: 11.2100x; 2.6024x over previous
import jax
import jax.numpy as jnp
from jax import lax
from jax.experimental import pallas as pl
from jax.experimental.pallas import tpu as pltpu
from jax.experimental.pallas import tpu_sc as plsc

N = 10000
D_IN = 128
D_H1 = 128
D_H2 = 64
E = 320000

NC = 2
NS = 16
NW = NC * NS
CW = 128
GB = 8
NGROUP = 10
NCHUNK = GB * NGROUP
EPAD = NW * NCHUNK * CW
SINK = N
NPAD = 10016
ZPT = NPAD // NS
DRT = N // NS

BLK = 1000


def _fill_zero(ref, rows, cols):
    vec = jnp.zeros((16,), dtype=jnp.float32)

    def body(r, _):
        for k in range(cols // 16):
            ref[r, pl.ds(k * 16, 16)] = vec
        return 0

    lax.fori_loop(0, rows, body, 0, unroll=False)


def _make_spmm(D, with_deg):
    mesh = plsc.VectorSubcoreMesh(core_axis_name="c", subcore_axis_name="s")

    out_type = [jax.ShapeDtypeStruct((NC * N, D), jnp.float32)]
    scratch = [
        pltpu.VMEM((GB, CW), jnp.int32),
        pltpu.VMEM((GB, CW), jnp.int32),
        pltpu.VMEM((CW, D), jnp.float32),
        pltpu.VMEM((CW, D), jnp.float32),
        pltpu.VMEM_SHARED((NPAD, D), jnp.float32),
        pltpu.SemaphoreType.DMA,
        pltpu.SemaphoreType.DMA,
    ]
    if with_deg:
        out_type.append(jax.ShapeDtypeStruct((NW * NPAD,), jnp.float32))
        scratch.append(pltpu.VMEM((NPAD,), jnp.float32))

    def body(y_hbm, srcs_hbm, dsts_hbm, *rest):
        if with_deg:
            (out_hbm, deg_hbm, src_v, dst_v, rows_a, rows_b, acc,
             sem_a, sem_b, hist) = rest
        else:
            (out_hbm, src_v, dst_v, rows_a, rows_b, acc,
             sem_a, sem_b) = rest
        cid = lax.axis_index("c")
        sid = lax.axis_index("s")
        wid = cid * NS + sid
        bufs = (rows_a, rows_b)
        sems = (sem_a, sem_b)
        ones16 = jnp.full((16,), 1.0, dtype=jnp.float32)

        _fill_zero(rows_a, CW, D)
        z0 = sid * ZPT
        zoff = 0
        while zoff < ZPT:
            zn = min(CW, ZPT - zoff)
            pltpu.sync_copy(rows_a.at[pl.ds(0, zn)],
                            acc.at[pl.ds(z0 + zoff, zn)])
            zoff += zn
        if with_deg:
            zv = jnp.zeros((16,), dtype=jnp.float32)

            def zrow(r, _):
                hist[pl.ds(r * 16, 16)] = zv
                return 0

            lax.fori_loop(0, NPAD // 16, zrow, 0, unroll=False)
        plsc.subcore_barrier()

        def group(g, _):
            i0 = wid * NCHUNK + g * GB
            pltpu.sync_copy(srcs_hbm.at[pl.ds(i0, GB)], src_v)
            pltpu.sync_copy(dsts_hbm.at[pl.ds(i0, GB)], dst_v)
            cp = pltpu.async_copy(y_hbm.at[src_v.at[0]], rows_a, sem_a)
            for b in range(GB):
                buf = bufs[b % 2]
                cp.wait()
                if b + 1 < GB:
                    cp = pltpu.async_copy(y_hbm.at[src_v.at[b + 1]],
                                          bufs[(b + 1) % 2],
                                          sems[(b + 1) % 2])
                if with_deg:
                    for l in range(CW // 16):
                        idx = dst_v[b, pl.ds(l * 16, 16)]
                        plsc.addupdate_scatter(hist, [idx], ones16)
                pltpu.sync_copy(buf, acc.at[dst_v.at[b]], add=True)
            return 0

        lax.fori_loop(0, NGROUP, group, 0, unroll=False)
        plsc.subcore_barrier()

        t0 = sid * DRT
        for q in range(5):
            pltpu.sync_copy(acc.at[pl.ds(t0 + q * 125, 125)],
                            rows_a.at[pl.ds(0, 125)])
            pltpu.sync_copy(rows_a.at[pl.ds(0, 125)],
                            out_hbm.at[pl.ds(cid * N + t0 + q * 125, 125)])
        if with_deg:
            pltpu.sync_copy(hist, deg_hbm.at[pl.ds(wid * NPAD, NPAD)])

    return pl.kernel(
        body, out_type=out_type, mesh=mesh, scratch_types=scratch,
        compiler_params=pltpu.CompilerParams(use_tc_tiling_on_sc=False,
                                             needs_layout_passes=False))


_spmm_deg = _make_spmm(D_H1, True)
_spmm = _make_spmm(D_H2, False)


def _pre_body(x_ref, wl_ref, wr_ref, b1_ref, y1_ref, z1_ref):
    x = x_ref[...]
    y1_ref[...] = jnp.dot(x, wl_ref[...], preferred_element_type=jnp.float32)
    z1_ref[...] = (jnp.dot(x, wr_ref[...], preferred_element_type=jnp.float32)
                   + b1_ref[...])


def _pre(x, W1l, W1r, b1):
    return pl.pallas_call(
        _pre_body,
        grid=(N // BLK,),
        in_specs=[
            pl.BlockSpec((BLK, D_IN), lambda i: (i, 0)),
            pl.BlockSpec((D_IN, D_H1), lambda i: (0, 0)),
            pl.BlockSpec((D_IN, D_H1), lambda i: (0, 0)),
            pl.BlockSpec((1, D_H1), lambda i: (0, 0)),
        ],
        out_specs=[
            pl.BlockSpec((BLK, D_H1), lambda i: (i, 0)),
            pl.BlockSpec((BLK, D_H1), lambda i: (i, 0)),
        ],
        out_shape=[
            jax.ShapeDtypeStruct((N, D_H1), jnp.float32),
            jax.ShapeDtypeStruct((N, D_H1), jnp.float32),
        ],
    )(x, W1l, W1r, b1)


def _rdeg(degt_ref):
    deg = jnp.sum(degt_ref[...], axis=1, keepdims=True)
    return 1.0 / jnp.maximum(deg, 1.0)


def _mid_body(s1a_ref, s1b_ref, degt_ref, z1_ref, w2l_ref, w2r_ref,
              b2_ref, y2_ref, z2_ref):
    s1 = s1a_ref[...] + s1b_ref[...]
    h1 = jnp.maximum(s1 * _rdeg(degt_ref) + z1_ref[...], 0.0)
    y2_ref[...] = jnp.dot(h1, w2l_ref[...], preferred_element_type=jnp.float32)
    z2_ref[...] = (jnp.dot(h1, w2r_ref[...], preferred_element_type=jnp.float32)
                   + b2_ref[...])


def _mid(s1p, degt, z1, W2l, W2r, b2):
    return pl.pallas_call(
        _mid_body,
        grid=(N // BLK,),
        in_specs=[
            pl.BlockSpec((BLK, D_H1), lambda i: (i, 0)),
            pl.BlockSpec((BLK, D_H1), lambda i: (i + N // BLK, 0)),
            pl.BlockSpec((BLK, NW), lambda i: (i, 0)),
            pl.BlockSpec((BLK, D_H1), lambda i: (i, 0)),
            pl.BlockSpec((D_H1, D_H2), lambda i: (0, 0)),
            pl.BlockSpec((D_H1, D_H2), lambda i: (0, 0)),
            pl.BlockSpec((1, D_H2), lambda i: (0, 0)),
        ],
        out_specs=[
            pl.BlockSpec((BLK, D_H2), lambda i: (i, 0)),
            pl.BlockSpec((BLK, D_H2), lambda i: (i, 0)),
        ],
        out_shape=[
            jax.ShapeDtypeStruct((N, D_H2), jnp.float32),
            jax.ShapeDtypeStruct((N, D_H2), jnp.float32),
        ],
    )(s1p, s1p, degt, z1, W2l, W2r, b2)


def _post_body(s2a_ref, s2b_ref, degt_ref, z2_ref, wc_ref, bc_ref, out_ref):
    mean2 = (s2a_ref[...] + s2b_ref[...]) * _rdeg(degt_ref)
    h2 = jnp.maximum(mean2 + z2_ref[...], 0.0)
    logits = (jnp.dot(h2, wc_ref[...], preferred_element_type=jnp.float32)
              + bc_ref[...])
    l0 = logits[:, 0:1]
    l1 = logits[:, 1:2]
    m = jnp.maximum(l0, l1)
    lse = m + jnp.log(jnp.exp(l0 - m) + jnp.exp(l1 - m))
    out_ref[...] = jnp.concatenate([l0 - lse, l1 - lse], axis=1)


def _post(s2p, degt, z2, Wc, bc):
    return pl.pallas_call(
        _post_body,
        grid=(N // BLK,),
        in_specs=[
            pl.BlockSpec((BLK, D_H2), lambda i: (i, 0)),
            pl.BlockSpec((BLK, D_H2), lambda i: (i + N // BLK, 0)),
            pl.BlockSpec((BLK, NW), lambda i: (i, 0)),
            pl.BlockSpec((BLK, D_H2), lambda i: (i, 0)),
            pl.BlockSpec((D_H2, 128), lambda i: (0, 0)),
            pl.BlockSpec((1, 128), lambda i: (0, 0)),
        ],
        out_specs=pl.BlockSpec((BLK, 2), lambda i: (i, 0)),
        out_shape=jax.ShapeDtypeStruct((N, 2), jnp.float32),
    )(s2p, s2p, degt, z2, Wc, bc)


def kernel(x, edge_index, W1l, W1r, b1, W2l, W2r, b2, Wc, bc):
    src = edge_index[0].astype(jnp.int32)
    dst = edge_index[1].astype(jnp.int32)
    pad = EPAD - E
    pad_src = jnp.arange(pad, dtype=jnp.int32) % 128
    pad_dst = SINK + jnp.arange(pad, dtype=jnp.int32) % (NPAD - SINK)
    srcs = jnp.concatenate([src, pad_src]).reshape(NW * NCHUNK, CW)
    dsts = jnp.concatenate([dst, pad_dst]).reshape(NW * NCHUNK, CW)

    wc_pad = jnp.zeros((D_H2, 128), jnp.float32).at[:, :2].set(Wc)
    bc_pad = jnp.zeros((1, 128), jnp.float32).at[0, :2].set(bc)

    y1, z1 = _pre(x, W1l, W1r, b1.reshape(1, -1))
    s1p, deg_flat = _spmm_deg(y1, srcs, dsts)
    degt = deg_flat.reshape(NW, NPAD).T
    y2, z2 = _mid(s1p, degt, z1, W2l, W2r, b2.reshape(1, -1))
    (s2p,) = _spmm(y2, srcs, dsts)
    return _post(s2p, degt, z2, wc_pad, bc_pad)


# speedup vs baseline: 13.1728x; 1.1751x over previous
import jax
import jax.numpy as jnp
from jax import lax
from jax.experimental import pallas as pl
from jax.experimental.pallas import tpu as pltpu
from jax.experimental.pallas import tpu_sc as plsc

N = 10000
D_IN = 128
D_H1 = 128
D_H2 = 64
E = 320000

NC = 2
NS = 16
NW = NC * NS
CW = 128
GB = 8
NGROUP = 10
NCHUNK = GB * NGROUP
EPAD = NW * NCHUNK * CW
SINK = N
NPAD = 10016
ZPT = NPAD // NS
DRT = N // NS

BLK = 1000


def _fill_zero(ref, rows, cols):
    vec = jnp.zeros((16,), dtype=jnp.float32)

    def body(r, _):
        for k in range(cols // 16):
            ref[r, pl.ds(k * 16, 16)] = vec
        return 0

    lax.fori_loop(0, rows, body, 0, unroll=False)


def _make_spmm(D, with_deg, gb, nbuf):
    mesh = plsc.VectorSubcoreMesh(core_axis_name="c", subcore_axis_name="s")
    ngroup = NCHUNK // gb

    out_type = [jax.ShapeDtypeStruct((NC * N, D), jnp.float32)]
    scratch = [
        pltpu.VMEM((gb, CW), jnp.int32),
        pltpu.VMEM((gb, CW), jnp.int32),
    ]
    scratch += [pltpu.VMEM((CW, D), jnp.float32) for _ in range(nbuf)]
    scratch.append(pltpu.VMEM_SHARED((NPAD, D), jnp.float32))
    scratch += [pltpu.SemaphoreType.DMA for _ in range(nbuf)]
    if with_deg:
        out_type.append(jax.ShapeDtypeStruct((NW * NPAD,), jnp.float32))
        scratch.append(pltpu.VMEM((NPAD,), jnp.float32))

    def body(y_hbm, srcs_hbm, dsts_hbm, *rest):
        if with_deg:
            out_hbm, deg_hbm = rest[0], rest[1]
            rest = rest[2:]
        else:
            out_hbm = rest[0]
            rest = rest[1:]
        src_v, dst_v = rest[0], rest[1]
        bufs = rest[2:2 + nbuf]
        acc = rest[2 + nbuf]
        sems = rest[3 + nbuf:3 + 2 * nbuf]
        hist = rest[3 + 2 * nbuf] if with_deg else None
        cid = lax.axis_index("c")
        sid = lax.axis_index("s")
        wid = cid * NS + sid
        rows_a = bufs[0]
        ones16 = jnp.full((16,), 1.0, dtype=jnp.float32)

        _fill_zero(rows_a, CW, D)
        z0 = sid * ZPT
        zoff = 0
        while zoff < ZPT:
            zn = min(CW, ZPT - zoff)
            pltpu.sync_copy(rows_a.at[pl.ds(0, zn)],
                            acc.at[pl.ds(z0 + zoff, zn)])
            zoff += zn
        if with_deg:
            zv = jnp.zeros((16,), dtype=jnp.float32)

            def zrow(r, _):
                hist[pl.ds(r * 16, 16)] = zv
                return 0

            lax.fori_loop(0, NPAD // 16, zrow, 0, unroll=False)
        plsc.subcore_barrier()

        def group(g, _):
            i0 = wid * NCHUNK + g * gb
            pltpu.sync_copy(srcs_hbm.at[pl.ds(i0, gb)], src_v)
            pltpu.sync_copy(dsts_hbm.at[pl.ds(i0, gb)], dst_v)
            cps = [pltpu.async_copy(y_hbm.at[src_v.at[b]], bufs[b], sems[b])
                   for b in range(min(nbuf, gb))]
            for b in range(gb):
                cps[b % nbuf].wait()
                if with_deg:
                    for l in range(CW // 16):
                        idx = dst_v[b, pl.ds(l * 16, 16)]
                        plsc.addupdate_scatter(hist, [idx], ones16)
                pltpu.sync_copy(bufs[b % nbuf], acc.at[dst_v.at[b]], add=True)
                if b + nbuf < gb:
                    cps[b % nbuf] = pltpu.async_copy(
                        y_hbm.at[src_v.at[b + nbuf]], bufs[b % nbuf],
                        sems[b % nbuf])
            return 0

        lax.fori_loop(0, ngroup, group, 0, unroll=False)
        plsc.subcore_barrier()

        t0 = sid * DRT
        for q in range(5):
            pltpu.sync_copy(acc.at[pl.ds(t0 + q * 125, 125)],
                            rows_a.at[pl.ds(0, 125)])
            pltpu.sync_copy(rows_a.at[pl.ds(0, 125)],
                            out_hbm.at[pl.ds(cid * N + t0 + q * 125, 125)])
        if with_deg:
            pltpu.sync_copy(hist, deg_hbm.at[pl.ds(wid * NPAD, NPAD)])

    return pl.kernel(
        body, out_type=out_type, mesh=mesh, scratch_types=scratch,
        compiler_params=pltpu.CompilerParams(use_tc_tiling_on_sc=False,
                                             needs_layout_passes=False))


_spmm_deg = _make_spmm(D_H1, True, gb=8, nbuf=2)
_spmm = _make_spmm(D_H2, False, gb=16, nbuf=4)


def _pre_body(x_ref, wl_ref, wr_ref, b1_ref, y1_ref, z1_ref):
    x = x_ref[...]
    y1_ref[...] = jnp.dot(x, wl_ref[...], preferred_element_type=jnp.float32)
    z1_ref[...] = (jnp.dot(x, wr_ref[...], preferred_element_type=jnp.float32)
                   + b1_ref[...])


def _pre(x, W1l, W1r, b1):
    return pl.pallas_call(
        _pre_body,
        grid=(N // BLK,),
        in_specs=[
            pl.BlockSpec((BLK, D_IN), lambda i: (i, 0)),
            pl.BlockSpec((D_IN, D_H1), lambda i: (0, 0)),
            pl.BlockSpec((D_IN, D_H1), lambda i: (0, 0)),
            pl.BlockSpec((1, D_H1), lambda i: (0, 0)),
        ],
        out_specs=[
            pl.BlockSpec((BLK, D_H1), lambda i: (i, 0)),
            pl.BlockSpec((BLK, D_H1), lambda i: (i, 0)),
        ],
        out_shape=[
            jax.ShapeDtypeStruct((N, D_H1), jnp.float32),
            jax.ShapeDtypeStruct((N, D_H1), jnp.float32),
        ],
    )(x, W1l, W1r, b1)


def _rdeg(degt_ref):
    deg = jnp.sum(degt_ref[...], axis=1, keepdims=True)
    return 1.0 / jnp.maximum(deg, 1.0)


def _mid_body(s1a_ref, s1b_ref, degt_ref, z1_ref, w2l_ref, w2r_ref,
              b2_ref, y2_ref, z2_ref):
    s1 = s1a_ref[...] + s1b_ref[...]
    h1 = jnp.maximum(s1 * _rdeg(degt_ref) + z1_ref[...], 0.0)
    y2_ref[...] = jnp.dot(h1, w2l_ref[...], preferred_element_type=jnp.float32)
    z2_ref[...] = (jnp.dot(h1, w2r_ref[...], preferred_element_type=jnp.float32)
                   + b2_ref[...])


def _mid(s1p, degt, z1, W2l, W2r, b2):
    return pl.pallas_call(
        _mid_body,
        grid=(N // BLK,),
        in_specs=[
            pl.BlockSpec((BLK, D_H1), lambda i: (i, 0)),
            pl.BlockSpec((BLK, D_H1), lambda i: (i + N // BLK, 0)),
            pl.BlockSpec((BLK, NW), lambda i: (i, 0)),
            pl.BlockSpec((BLK, D_H1), lambda i: (i, 0)),
            pl.BlockSpec((D_H1, D_H2), lambda i: (0, 0)),
            pl.BlockSpec((D_H1, D_H2), lambda i: (0, 0)),
            pl.BlockSpec((1, D_H2), lambda i: (0, 0)),
        ],
        out_specs=[
            pl.BlockSpec((BLK, D_H2), lambda i: (i, 0)),
            pl.BlockSpec((BLK, D_H2), lambda i: (i, 0)),
        ],
        out_shape=[
            jax.ShapeDtypeStruct((N, D_H2), jnp.float32),
            jax.ShapeDtypeStruct((N, D_H2), jnp.float32),
        ],
    )(s1p, s1p, degt, z1, W2l, W2r, b2)


def _post_body(s2a_ref, s2b_ref, degt_ref, z2_ref, wc_ref, bc_ref, out_ref):
    mean2 = (s2a_ref[...] + s2b_ref[...]) * _rdeg(degt_ref)
    h2 = jnp.maximum(mean2 + z2_ref[...], 0.0)
    logits = (jnp.dot(h2, wc_ref[...], preferred_element_type=jnp.float32)
              + bc_ref[...])
    l0 = logits[:, 0:1]
    l1 = logits[:, 1:2]
    m = jnp.maximum(l0, l1)
    lse = m + jnp.log(jnp.exp(l0 - m) + jnp.exp(l1 - m))
    out_ref[...] = jnp.concatenate([l0 - lse, l1 - lse], axis=1)


def _post(s2p, degt, z2, Wc, bc):
    return pl.pallas_call(
        _post_body,
        grid=(N // BLK,),
        in_specs=[
            pl.BlockSpec((BLK, D_H2), lambda i: (i, 0)),
            pl.BlockSpec((BLK, D_H2), lambda i: (i + N // BLK, 0)),
            pl.BlockSpec((BLK, NW), lambda i: (i, 0)),
            pl.BlockSpec((BLK, D_H2), lambda i: (i, 0)),
            pl.BlockSpec((D_H2, 128), lambda i: (0, 0)),
            pl.BlockSpec((1, 128), lambda i: (0, 0)),
        ],
        out_specs=pl.BlockSpec((BLK, 2), lambda i: (i, 0)),
        out_shape=jax.ShapeDtypeStruct((N, 2), jnp.float32),
    )(s2p, s2p, degt, z2, Wc, bc)


def kernel(x, edge_index, W1l, W1r, b1, W2l, W2r, b2, Wc, bc):
    src = edge_index[0].astype(jnp.int32)
    dst = edge_index[1].astype(jnp.int32)
    pad = EPAD - E
    pad_src = jnp.arange(pad, dtype=jnp.int32) % 128
    pad_dst = SINK + jnp.arange(pad, dtype=jnp.int32) % (NPAD - SINK)
    srcs = jnp.concatenate([src, pad_src]).reshape(NW * NCHUNK, CW)
    dsts = jnp.concatenate([dst, pad_dst]).reshape(NW * NCHUNK, CW)

    wc_pad = jnp.zeros((D_H2, 128), jnp.float32).at[:, :2].set(Wc)
    bc_pad = jnp.zeros((1, 128), jnp.float32).at[0, :2].set(bc)

    y1, z1 = _pre(x, W1l, W1r, b1.reshape(1, -1))
    s1p, deg_flat = _spmm_deg(y1, srcs, dsts)
    degt = deg_flat.reshape(NW, NPAD).T
    y2, z2 = _mid(s1p, degt, z1, W2l, W2r, b2.reshape(1, -1))
    (s2p,) = _spmm(y2, srcs, dsts)
    return _post(s2p, degt, z2, wc_pad, bc_pad)


# speedup vs baseline: 13.2424x; 1.0053x over previous
import jax
import jax.numpy as jnp
from jax import lax
from jax.experimental import pallas as pl
from jax.experimental.pallas import tpu as pltpu
from jax.experimental.pallas import tpu_sc as plsc

N = 10000
D_IN = 128
D_H1 = 128
D_H2 = 64
E = 320000

NC = 2
NS = 16
NW = NC * NS
CW = 128
GB = 8
NGROUP = 10
NCHUNK = GB * NGROUP
EPAD = NW * NCHUNK * CW
SINK = N
NPAD = 10016
ZPT = NPAD // NS
DRT = N // NS

BLK = 1000


def _fill_zero(ref, rows, cols):
    vec = jnp.zeros((16,), dtype=jnp.float32)

    def body(r, _):
        for k in range(cols // 16):
            ref[r, pl.ds(k * 16, 16)] = vec
        return 0

    lax.fori_loop(0, rows, body, 0, unroll=False)


def _make_spmm(D, with_deg, gb, nbuf):
    mesh = plsc.VectorSubcoreMesh(core_axis_name="c", subcore_axis_name="s")
    ngroup = NCHUNK // gb

    out_type = [jax.ShapeDtypeStruct((NC * N, D), jnp.float32)]
    scratch = [
        pltpu.VMEM((gb, CW), jnp.int32),
        pltpu.VMEM((gb, CW), jnp.int32),
    ]
    scratch += [pltpu.VMEM((CW, D), jnp.float32) for _ in range(nbuf)]
    scratch.append(pltpu.VMEM_SHARED((NPAD, D), jnp.float32))
    scratch += [pltpu.SemaphoreType.DMA for _ in range(2 * nbuf)]
    if with_deg:
        out_type.append(jax.ShapeDtypeStruct((NW * NPAD,), jnp.float32))
        scratch.append(pltpu.VMEM((NPAD,), jnp.float32))

    def body(y_hbm, srcs_hbm, dsts_hbm, *rest):
        if with_deg:
            out_hbm, deg_hbm = rest[0], rest[1]
            rest = rest[2:]
        else:
            out_hbm = rest[0]
            rest = rest[1:]
        src_v, dst_v = rest[0], rest[1]
        bufs = rest[2:2 + nbuf]
        acc = rest[2 + nbuf]
        sems = rest[3 + nbuf:3 + 2 * nbuf]
        ssems = rest[3 + 2 * nbuf:3 + 3 * nbuf]
        hist = rest[3 + 3 * nbuf] if with_deg else None
        cid = lax.axis_index("c")
        sid = lax.axis_index("s")
        wid = cid * NS + sid
        rows_a = bufs[0]
        ones16 = jnp.full((16,), 1.0, dtype=jnp.float32)

        _fill_zero(rows_a, CW, D)
        z0 = sid * ZPT
        zoff = 0
        while zoff < ZPT:
            zn = min(CW, ZPT - zoff)
            pltpu.sync_copy(rows_a.at[pl.ds(0, zn)],
                            acc.at[pl.ds(z0 + zoff, zn)])
            zoff += zn
        if with_deg:
            zv = jnp.zeros((16,), dtype=jnp.float32)

            def zrow(r, _):
                hist[pl.ds(r * 16, 16)] = zv
                return 0

            lax.fori_loop(0, NPAD // 16, zrow, 0, unroll=False)
        plsc.subcore_barrier()

        def group(g, _):
            i0 = wid * NCHUNK + g * gb
            pltpu.sync_copy(srcs_hbm.at[pl.ds(i0, gb)], src_v)
            pltpu.sync_copy(dsts_hbm.at[pl.ds(i0, gb)], dst_v)
            lead = min(nbuf - 1, gb)
            cps = [pltpu.async_copy(y_hbm.at[src_v.at[b]], bufs[b % nbuf],
                                    sems[b % nbuf]) for b in range(lead)]
            scps = [None] * gb
            for b in range(gb):
                if b + lead < gb:
                    if b >= 1:
                        scps[b - 1].wait()
                    cps.append(pltpu.async_copy(
                        y_hbm.at[src_v.at[b + lead]], bufs[(b + lead) % nbuf],
                        sems[(b + lead) % nbuf]))
                cps[b].wait()
                if with_deg:
                    for l in range(CW // 16):
                        idx = dst_v[b, pl.ds(l * 16, 16)]
                        plsc.addupdate_scatter(hist, [idx], ones16)
                scps[b] = pltpu.async_copy(bufs[b % nbuf],
                                           acc.at[dst_v.at[b]],
                                           ssems[b % nbuf], add=True)
            for b in range(max(0, gb - lead - 1), gb):
                scps[b].wait()
            return 0

        lax.fori_loop(0, ngroup, group, 0, unroll=False)
        plsc.subcore_barrier()

        t0 = sid * DRT
        for q in range(5):
            pltpu.sync_copy(acc.at[pl.ds(t0 + q * 125, 125)],
                            rows_a.at[pl.ds(0, 125)])
            pltpu.sync_copy(rows_a.at[pl.ds(0, 125)],
                            out_hbm.at[pl.ds(cid * N + t0 + q * 125, 125)])
        if with_deg:
            pltpu.sync_copy(hist, deg_hbm.at[pl.ds(wid * NPAD, NPAD)])

    return pl.kernel(
        body, out_type=out_type, mesh=mesh, scratch_types=scratch,
        compiler_params=pltpu.CompilerParams(use_tc_tiling_on_sc=False,
                                             needs_layout_passes=False))


_spmm_deg = _make_spmm(D_H1, True, gb=8, nbuf=2)
_spmm = _make_spmm(D_H2, False, gb=16, nbuf=4)


def _pre_body(x_ref, wl_ref, wr_ref, b1_ref, y1_ref, z1_ref):
    x = x_ref[...]
    y1_ref[...] = jnp.dot(x, wl_ref[...], preferred_element_type=jnp.float32)
    z1_ref[...] = (jnp.dot(x, wr_ref[...], preferred_element_type=jnp.float32)
                   + b1_ref[...])


def _pre(x, W1l, W1r, b1):
    return pl.pallas_call(
        _pre_body,
        grid=(N // BLK,),
        in_specs=[
            pl.BlockSpec((BLK, D_IN), lambda i: (i, 0)),
            pl.BlockSpec((D_IN, D_H1), lambda i: (0, 0)),
            pl.BlockSpec((D_IN, D_H1), lambda i: (0, 0)),
            pl.BlockSpec((1, D_H1), lambda i: (0, 0)),
        ],
        out_specs=[
            pl.BlockSpec((BLK, D_H1), lambda i: (i, 0)),
            pl.BlockSpec((BLK, D_H1), lambda i: (i, 0)),
        ],
        out_shape=[
            jax.ShapeDtypeStruct((N, D_H1), jnp.float32),
            jax.ShapeDtypeStruct((N, D_H1), jnp.float32),
        ],
    )(x, W1l, W1r, b1)


def _rdeg(degt_ref):
    deg = jnp.sum(degt_ref[...], axis=1, keepdims=True)
    return 1.0 / jnp.maximum(deg, 1.0)


def _mid_body(s1a_ref, s1b_ref, degt_ref, z1_ref, w2l_ref, w2r_ref,
              b2_ref, y2_ref, z2_ref):
    s1 = s1a_ref[...] + s1b_ref[...]
    h1 = jnp.maximum(s1 * _rdeg(degt_ref) + z1_ref[...], 0.0)
    y2_ref[...] = jnp.dot(h1, w2l_ref[...], preferred_element_type=jnp.float32)
    z2_ref[...] = (jnp.dot(h1, w2r_ref[...], preferred_element_type=jnp.float32)
                   + b2_ref[...])


def _mid(s1p, degt, z1, W2l, W2r, b2):
    return pl.pallas_call(
        _mid_body,
        grid=(N // BLK,),
        in_specs=[
            pl.BlockSpec((BLK, D_H1), lambda i: (i, 0)),
            pl.BlockSpec((BLK, D_H1), lambda i: (i + N // BLK, 0)),
            pl.BlockSpec((BLK, NW), lambda i: (i, 0)),
            pl.BlockSpec((BLK, D_H1), lambda i: (i, 0)),
            pl.BlockSpec((D_H1, D_H2), lambda i: (0, 0)),
            pl.BlockSpec((D_H1, D_H2), lambda i: (0, 0)),
            pl.BlockSpec((1, D_H2), lambda i: (0, 0)),
        ],
        out_specs=[
            pl.BlockSpec((BLK, D_H2), lambda i: (i, 0)),
            pl.BlockSpec((BLK, D_H2), lambda i: (i, 0)),
        ],
        out_shape=[
            jax.ShapeDtypeStruct((N, D_H2), jnp.float32),
            jax.ShapeDtypeStruct((N, D_H2), jnp.float32),
        ],
    )(s1p, s1p, degt, z1, W2l, W2r, b2)


def _post_body(s2a_ref, s2b_ref, degt_ref, z2_ref, wc_ref, bc_ref, out_ref):
    mean2 = (s2a_ref[...] + s2b_ref[...]) * _rdeg(degt_ref)
    h2 = jnp.maximum(mean2 + z2_ref[...], 0.0)
    logits = (jnp.dot(h2, wc_ref[...], preferred_element_type=jnp.float32)
              + bc_ref[...])
    l0 = logits[:, 0:1]
    l1 = logits[:, 1:2]
    m = jnp.maximum(l0, l1)
    lse = m + jnp.log(jnp.exp(l0 - m) + jnp.exp(l1 - m))
    out_ref[...] = jnp.concatenate([l0 - lse, l1 - lse], axis=1)


def _post(s2p, degt, z2, Wc, bc):
    return pl.pallas_call(
        _post_body,
        grid=(N // BLK,),
        in_specs=[
            pl.BlockSpec((BLK, D_H2), lambda i: (i, 0)),
            pl.BlockSpec((BLK, D_H2), lambda i: (i + N // BLK, 0)),
            pl.BlockSpec((BLK, NW), lambda i: (i, 0)),
            pl.BlockSpec((BLK, D_H2), lambda i: (i, 0)),
            pl.BlockSpec((D_H2, 128), lambda i: (0, 0)),
            pl.BlockSpec((1, 128), lambda i: (0, 0)),
        ],
        out_specs=pl.BlockSpec((BLK, 2), lambda i: (i, 0)),
        out_shape=jax.ShapeDtypeStruct((N, 2), jnp.float32),
    )(s2p, s2p, degt, z2, Wc, bc)


def kernel(x, edge_index, W1l, W1r, b1, W2l, W2r, b2, Wc, bc):
    src = edge_index[0].astype(jnp.int32)
    dst = edge_index[1].astype(jnp.int32)
    pad = EPAD - E
    pad_src = jnp.arange(pad, dtype=jnp.int32) % 128
    pad_dst = SINK + jnp.arange(pad, dtype=jnp.int32) % (NPAD - SINK)
    srcs = jnp.concatenate([src, pad_src]).reshape(NW * NCHUNK, CW)
    dsts = jnp.concatenate([dst, pad_dst]).reshape(NW * NCHUNK, CW)

    wc_pad = jnp.zeros((D_H2, 128), jnp.float32).at[:, :2].set(Wc)
    bc_pad = jnp.zeros((1, 128), jnp.float32).at[0, :2].set(bc)

    y1, z1 = _pre(x, W1l, W1r, b1.reshape(1, -1))
    s1p, deg_flat = _spmm_deg(y1, srcs, dsts)
    degt = deg_flat.reshape(NW, NPAD).T
    y2, z2 = _mid(s1p, degt, z1, W2l, W2r, b2.reshape(1, -1))
    (s2p,) = _spmm(y2, srcs, dsts)
    return _post(s2p, degt, z2, wc_pad, bc_pad)
